# baseline (speedup 1.0000x reference)
import jax
import jax.numpy as jnp
from jax.experimental import pallas as pl


def _dense_head(h2, W3, b3, W4, b4, W5, b5):
    N = h2.shape[0]
    B = 10000

    def body(h2_ref, W3t_ref, b3_ref, W4t_ref, b4_ref, W5t_ref, out_ref):
        h2b = h2_ref[...]
        h3 = jnp.maximum(h2b @ W3t_ref[...] + b3_ref[...], 0.0)
        h4 = jnp.maximum(h3 @ W4t_ref[...] + b4_ref[...], 0.0)
        out_ref[...] = h4 @ W5t_ref[...]

    grid = (N // B,)
    out = pl.pallas_call(
        body,
        grid=grid,
        in_specs=[
            pl.BlockSpec((B, 16), lambda i: (i, 0)),
            pl.BlockSpec((16, 8), lambda i: (0, 0)),
            pl.BlockSpec((1, 8), lambda i: (0, 0)),
            pl.BlockSpec((8, 5), lambda i: (0, 0)),
            pl.BlockSpec((1, 5), lambda i: (0, 0)),
            pl.BlockSpec((5, 1), lambda i: (0, 0)),
        ],
        out_specs=pl.BlockSpec((B, 1), lambda i: (i, 0)),
        out_shape=jax.ShapeDtypeStruct((N, 1), jnp.float32),
    )(h2, W3.T, b3.reshape(1, 8), W4.T, b4.reshape(1, 5), W5.T)
    return out[:, 0] + b5[0]


def kernel(x, edge_index, norm, W1l, b1l, W1r, Wm1l, bm1l, Wm1r, W2l, b2l, W2r,
           Wm2l, bm2l, Wm2r, W3, b3, W4, b4, W5, b5):
    src = edge_index[0]
    dst = edge_index[1]
    n = x.shape[0]

    def wsage(h, Wl, bl, Wr):
        msg = h[src] * norm[:, None]
        agg = jax.ops.segment_sum(msg, dst, num_segments=n)
        return agg @ Wl.T + bl + h @ Wr.T

    def sage_max(h, Wl, bl, Wr):
        agg = jax.ops.segment_max(h[src], dst, num_segments=n)
        agg = jnp.where(jnp.isfinite(agg), agg, 0.0)
        return agg @ Wl.T + bl + h @ Wr.T

    y = jax.nn.relu(wsage(x, W1l, b1l, W1r))
    z = jax.nn.relu(sage_max(x, Wm1l, bm1l, Wm1r))
    h = jnp.concatenate([y, z], axis=1)
    y2 = jax.nn.relu(wsage(h, W2l, b2l, W2r))
    z2 = jax.nn.relu(sage_max(h, Wm2l, bm2l, Wm2r))
    h2 = jnp.concatenate([y2, z2], axis=1)
    return _dense_head(h2, W3, b3, W4, b4, W5, b5)


# speedup vs baseline: 1.2406x; 1.2406x over previous
import functools

import jax
import jax.numpy as jnp
from jax import lax
from jax.experimental import pallas as pl
from jax.experimental.pallas import tpu as pltpu
from jax.experimental.pallas import tpu_sc as plsc

_N = 100000
_E = 1600000
_NC = 2
_NS = 16
_NW = _NC * _NS
_EPT = _E // _NW
_NP = 100096
_RPT = _NP // _NS
_NO = 3136

_mesh = plsc.VectorSubcoreMesh(core_axis_name="c", subcore_axis_name="s")
_params = pltpu.CompilerParams(needs_layout_passes=False)


def _seg_sum(xf, src, dst, norm, F, CS):
    SH = F.bit_length() - 1

    @functools.partial(
        pl.kernel,
        out_type=jax.ShapeDtypeStruct((_NC * _NP * F,), jnp.float32),
        mesh=_mesh,
        compiler_params=_params,
        scratch_types=[
            pltpu.VMEM_SHARED((_NP * F,), jnp.float32),
            pltpu.VMEM((CS,), jnp.int32),
            pltpu.VMEM((CS,), jnp.int32),
            pltpu.VMEM((CS,), jnp.float32),
            pltpu.VMEM((CS * F,), jnp.float32),
            pltpu.VMEM((CS * F,), jnp.int32),
            pltpu.VMEM((_RPT,), jnp.float32),
            pltpu.SemaphoreType.DMA,
        ],
    )
    def k(x_hbm, src_hbm, dst_hbm, norm_hbm, out_hbm,
          acc, srcb, dstb, normb, rows, idxb, stage, sem):
        c = lax.axis_index("c")
        s = lax.axis_index("s")
        w = c * _NS + s
        iota = lax.iota(jnp.int32, 16)
        QW = _RPT

        def zero16(j, _):
            stage[pl.ds(j * 16, 16)] = jnp.zeros((16,), jnp.float32)
            return 0

        lax.fori_loop(0, QW // 16, zero16, 0)

        def zcopy(q, _):
            pltpu.sync_copy(stage, acc.at[pl.ds(s * _RPT * F + q * QW, QW)])
            return 0

        lax.fori_loop(0, F, zcopy, 0)
        plsc.subcore_barrier()

        def chunk(kk, _):
            base = w * _EPT + kk * CS
            pltpu.sync_copy(src_hbm.at[pl.ds(base, CS)], srcb)
            pltpu.sync_copy(dst_hbm.at[pl.ds(base, CS)], dstb)
            pltpu.sync_copy(norm_hbm.at[pl.ds(base, CS)], normb)

            def mkidx(j, _):
                t = iota + j * 16
                sv = plsc.load_gather(srcb, [t >> SH])
                idxb[pl.ds(j * 16, 16)] = sv * F + (t & (F - 1))
                return 0

            lax.fori_loop(0, (CS * F) // 16, mkidx, 0)
            pltpu.async_copy(x_hbm.at[idxb], rows, sem).wait()

            def mulnorm(j, _):
                t = iota + j * 16
                nv = plsc.load_gather(normb, [t >> SH])
                rows[pl.ds(j * 16, 16)] = rows[pl.ds(j * 16, 16)] * nv
                return 0

            lax.fori_loop(0, (CS * F) // 16, mulnorm, 0)

            def mkdidx(j, _):
                t = iota + j * 16
                dv = plsc.load_gather(dstb, [t >> SH])
                idxb[pl.ds(j * 16, 16)] = dv * F + (t & (F - 1))
                return 0

            lax.fori_loop(0, (CS * F) // 16, mkdidx, 0)
            pltpu.sync_copy(rows, acc.at[idxb], add=True)
            return 0

        lax.fori_loop(0, _EPT // CS, chunk, 0)
        plsc.subcore_barrier()

        def wb(q, _):
            pltpu.sync_copy(acc.at[pl.ds(s * _RPT * F + q * QW, QW)], stage)
            pltpu.sync_copy(
                stage,
                out_hbm.at[pl.ds(c * _NP * F + s * _RPT * F + q * QW, QW)])
            return 0

        lax.fori_loop(0, F, wb, 0)

    return k(xf, src, dst, norm)


def _seg_max(xf, src, dst, F, CM, BE, init, sel_inf):
    ACCW = _NO * F
    SH = F.bit_length() - 1

    @functools.partial(
        pl.kernel,
        out_type=jax.ShapeDtypeStruct((_NW * ACCW,), jnp.float32),
        mesh=_mesh,
        compiler_params=_params,
        scratch_types=[
            pltpu.VMEM((ACCW + 32,), jnp.float32),
            pltpu.VMEM((CM,), jnp.int32),
            pltpu.VMEM((CM,), jnp.int32),
            pltpu.VMEM((CM + 32,), jnp.int32),
            pltpu.VMEM((CM + 32,), jnp.int32),
            pltpu.VMEM((BE * F,), jnp.int32),
            pltpu.VMEM((BE * F + 16,), jnp.float32),
            pltpu.SemaphoreType.DMA,
        ],
    )
    def k(x_hbm, src_hbm, dst_hbm, out_hbm,
          acc, srcb, dstb, msrc, mdl, idxb, rows, sem):
        c = lax.axis_index("c")
        s = lax.axis_index("s")
        w = c * _NS + s
        iota = lax.iota(jnp.int32, 16)

        def ainit(j, _):
            acc[pl.ds(j * 16, 16)] = jnp.full((16,), init, jnp.float32)
            return 0

        lax.fori_loop(0, (ACCW + 32) // 16, ainit, 0)

        def pfill(j, _):
            msrc[pl.ds(j * 16, 16)] = (iota + j * 16) & 8191
            return 0

        lax.fori_loop(0, (CM + 32) // 16, pfill, 0)

        def chunk(kk, _):
            base = kk * CM
            pltpu.sync_copy(src_hbm.at[pl.ds(base, CM)], srcb)
            pltpu.sync_copy(dst_hbm.at[pl.ds(base, CM)], dstb)

            def scan16(j, cntv):
                dv = dstb[pl.ds(j * 16, 16)]
                sv = srcb[pl.ds(j * 16, 16)]
                m = (dv & 31) == w
                rank = plsc.cumsum(jnp.where(m, 1, 0).astype(jnp.int32))
                pos = cntv + rank - 1
                plsc.store_scatter(msrc, [pos], sv, mask=m)
                plsc.store_scatter(mdl, [pos], dv >> 5, mask=m)
                return cntv + plsc.all_reduce_population_count(m)

            cntv = lax.fori_loop(0, CM // 16, scan16,
                                 jnp.zeros((16,), jnp.int32))
            cnt = cntv[0]

            def flush(off):
                def mkidx(j, _):
                    t = iota + j * 16
                    iv = jnp.minimum(off + (t >> SH), cnt - 1)
                    sv = plsc.load_gather(msrc, [iv])
                    idxb[pl.ds(j * 16, 16)] = sv * F + (t & (F - 1))
                    return 0

                lax.fori_loop(0, (BE * F) // 16, mkidx, 0)
                pltpu.async_copy(x_hbm.at[idxb], rows.at[pl.ds(0, BE * F)],
                                 sem).wait()
                nloc = jnp.minimum(BE, cnt - off)

                def accum(e, _):
                    dl = plsc.load_gather(
                        mdl, [jnp.full((16,), off + e, jnp.int32)])[0]
                    if F == 2:
                        ridx = e * 2 + iota
                        aidx = dl * 2 + iota
                        r = plsc.load_gather(rows, [ridx])
                        a = plsc.load_gather(acc, [aidx])
                        mx = jnp.maximum(a, r)
                        plsc.store_scatter(acc, [aidx], mx, mask=iota < 2)
                    else:
                        r0 = rows[pl.ds(e * 32, 16)]
                        a0 = acc[pl.ds(dl * 32, 16)]
                        acc[pl.ds(dl * 32, 16)] = jnp.maximum(a0, r0)
                        r1 = rows[pl.ds(e * 32 + 16, 16)]
                        a1 = acc[pl.ds(dl * 32 + 16, 16)]
                        acc[pl.ds(dl * 32 + 16, 16)] = jnp.maximum(a1, r1)
                    return 0

                lax.fori_loop(0, nloc, accum, 0)
                return off + BE

            lax.while_loop(lambda off: off < cnt, flush, jnp.int32(0))
            return 0

        lax.fori_loop(0, _E // CM, chunk, 0)

        if sel_inf:
            def fixinf(j, _):
                v = acc[pl.ds(j * 16, 16)]
                acc[pl.ds(j * 16, 16)] = jnp.where(v == init, 0.0, v)
                return 0

            lax.fori_loop(0, ACCW // 16, fixinf, 0)

        pltpu.sync_copy(acc.at[pl.ds(0, ACCW)],
                        out_hbm.at[pl.ds(w * ACCW, ACCW)])

    return k(xf, src, dst)


def _tc_layer1(x, s0, s1, mx, W1lT, b1l, W1rT, Wm1lT, bm1l, Wm1rT, W2lpT):
    B = 4000

    def body(x_r, s0_r, s1_r, mx_r, w1l_r, b1l_r, w1r_r, wm1l_r, bm1l_r,
             wm1r_r, w2lp_r, h_r, p_r):
        xb = x_r[...]
        agg = s0_r[...] + s1_r[...]
        y = jnp.maximum(agg @ w1l_r[...] + b1l_r[...] + xb @ w1r_r[...], 0.0)
        z = jnp.maximum(
            mx_r[...] @ wm1l_r[...] + bm1l_r[...] + xb @ wm1r_r[...], 0.0)
        h = jnp.concatenate([y, z], axis=1)
        h_r[...] = h
        p_r[...] = h @ w2lp_r[...]

    reps = lambda shp: pl.BlockSpec(shp, lambda i: (0, 0))
    h, p = pl.pallas_call(
        body,
        grid=(_N // B,),
        in_specs=[
            pl.BlockSpec((B, 2), lambda i: (i, 0)),
            pl.BlockSpec((B, 2), lambda i: (i, 0)),
            pl.BlockSpec((B, 2), lambda i: (i, 0)),
            pl.BlockSpec((B, 2), lambda i: (i, 0)),
            reps((2, 24)), reps((1, 24)), reps((2, 24)),
            reps((2, 8)), reps((1, 8)), reps((2, 8)),
            reps((32, 16)),
        ],
        out_specs=[pl.BlockSpec((B, 32), lambda i: (i, 0)),
                   pl.BlockSpec((B, 16), lambda i: (i, 0))],
        out_shape=[jax.ShapeDtypeStruct((_N, 32), jnp.float32),
                   jax.ShapeDtypeStruct((_N, 16), jnp.float32)],
    )(x, s0, s1, mx, W1lT, b1l, W1rT, Wm1lT, bm1l, Wm1rT, W2lpT)
    return h, p


def _tc_layer2(h, s0, s1, mx2, b2l, W2rT, Wm2lT, bm2l, Wm2rT, W3T, b3, W4T,
               b4, W5T):
    B = 4000

    def body(h_r, s0_r, s1_r, mx_r, b2l_r, w2r_r, wm2l_r, bm2l_r, wm2r_r,
             w3_r, b3_r, w4_r, b4_r, w5_r, o_r):
        hb = h_r[...]
        sp = (s0_r[...] + s1_r[...])[:, :12]
        y2 = jnp.maximum(sp + b2l_r[...] + hb @ w2r_r[...], 0.0)
        z2 = jnp.maximum(
            mx_r[...] @ wm2l_r[...] + bm2l_r[...] + hb @ wm2r_r[...], 0.0)
        h2 = jnp.concatenate([y2, z2], axis=1)
        h3 = jnp.maximum(h2 @ w3_r[...] + b3_r[...], 0.0)
        h4 = jnp.maximum(h3 @ w4_r[...] + b4_r[...], 0.0)
        o_r[...] = h4 @ w5_r[...]

    reps = lambda shp: pl.BlockSpec(shp, lambda i: (0, 0))
    out = pl.pallas_call(
        body,
        grid=(_N // B,),
        in_specs=[
            pl.BlockSpec((B, 32), lambda i: (i, 0)),
            pl.BlockSpec((B, 16), lambda i: (i, 0)),
            pl.BlockSpec((B, 16), lambda i: (i, 0)),
            pl.BlockSpec((B, 32), lambda i: (i, 0)),
            reps((1, 12)), reps((32, 12)),
            reps((32, 4)), reps((1, 4)), reps((32, 4)),
            reps((16, 8)), reps((1, 8)),
            reps((8, 5)), reps((1, 5)),
            reps((5, 1)),
        ],
        out_specs=pl.BlockSpec((B, 1), lambda i: (i, 0)),
        out_shape=jax.ShapeDtypeStruct((_N, 1), jnp.float32),
    )(h, s0, s1, mx2, b2l.reshape(1, 12), W2rT, Wm2lT, bm2l.reshape(1, 4),
      Wm2rT, W3T, b3.reshape(1, 8), W4T, b4.reshape(1, 5), W5T)
    return out


def kernel(x, edge_index, norm, W1l, b1l, W1r, Wm1l, bm1l, Wm1r, W2l, b2l, W2r,
           Wm2l, bm2l, Wm2r, W3, b3, W4, b4, W5, b5):
    src = edge_index[0]
    dst = edge_index[1]

    sp1 = _seg_sum(x.reshape(-1), src, dst, norm, 2, 2000)
    sp1 = sp1.reshape(_NC, _NP, 2)
    mx1 = _seg_max(x.reshape(-1), src, dst, 2, 12800, 256,
                   float(jnp.finfo(jnp.float32).min), True)
    mx1 = mx1.reshape(_NW, _NO, 2)[:, :3125]
    mx1 = jnp.transpose(mx1, (1, 0, 2)).reshape(_N, 2)

    W2lp = jnp.concatenate([W2l, jnp.zeros((4, 32), jnp.float32)], axis=0)
    h, p = _tc_layer1(x, sp1[0, :_N], sp1[1, :_N], mx1,
                      W1l.T, b1l.reshape(1, 24), W1r.T,
                      Wm1l.T, bm1l.reshape(1, 8), Wm1r.T, W2lp.T)

    sp2 = _seg_sum(p.reshape(-1), src, dst, norm, 16, 400)
    sp2 = sp2.reshape(_NC, _NP, 16)
    mx2 = _seg_max(h.reshape(-1), src, dst, 32, 4000, 128, 0.0, False)
    mx2 = mx2.reshape(_NW, _NO, 32)[:, :3125]
    mx2 = jnp.transpose(mx2, (1, 0, 2)).reshape(_N, 32)

    out = _tc_layer2(h, sp2[0, :_N], sp2[1, :_N], mx2,
                     b2l, W2r.T, Wm2l.T, bm2l, Wm2r.T,
                     W3.T, b3, W4.T, b4, W5.T)
    return out[:, 0] + b5[0]
